# baseline (speedup 1.0000x reference)
import functools

import jax
import jax.numpy as jnp
from jax import lax
from jax.experimental import pallas as pl
from jax.experimental.pallas import tpu as pltpu
from jax.experimental.pallas import tpu_sc as plsc

N_REL = 20000
N_PAD = 20480
N_OBJ = 5000
C_REL = 51
C_OBJ = 151
W = 64
BIG = 10**9


def _obj_body(obj_ref, d_ref, score_ref, label_ref):
    x = obj_ref[...]
    m = jnp.max(x, axis=1, keepdims=True)
    x1 = x[:, 1:]
    m1 = jnp.max(x1, axis=1, keepdims=True)
    score_ref[...] = jnp.exp(m1 - m) / d_ref[...]
    iota = lax.broadcasted_iota(jnp.int32, x1.shape, 1)
    cand = jnp.where(x1 == m1, iota, BIG)
    label_ref[...] = jnp.min(cand, axis=1, keepdims=True) + 1


def _tc_obj(obj_logit, dobj):
    return pl.pallas_call(
        _obj_body,
        out_shape=(
            jax.ShapeDtypeStruct((N_OBJ, 1), jnp.float32),
            jax.ShapeDtypeStruct((N_OBJ, 1), jnp.int32),
        ),
    )(obj_logit, dobj)


_REL_BLK = 2048


def _rel_body(rel_ref, pair_ref, d_ref, comb_ref, rs_ref):
    x = rel_ref[...]
    m = jnp.max(x, axis=1, keepdims=True)
    e = jnp.exp(x - m)
    p = e / d_ref[...]
    rs_ref[...] = jnp.max(p[:, 1:], axis=1, keepdims=True)
    pm = jnp.max(p, axis=1, keepdims=True)
    iota = lax.broadcasted_iota(jnp.int32, p.shape, 1)
    cls = jnp.min(jnp.where(p == pm, iota, BIG), axis=1, keepdims=True)
    pairf = pair_ref[...].astype(jnp.float32)
    comb_ref[...] = jnp.concatenate(
        [p, cls.astype(jnp.float32), pairf,
         jnp.zeros((x.shape[0], W - C_REL - 3), jnp.float32)], axis=1)


def _tc_rel(rel_pad, pair_pad, drel):
    grid = N_PAD // _REL_BLK
    return pl.pallas_call(
        _rel_body,
        grid=(grid,),
        in_specs=[
            pl.BlockSpec((_REL_BLK, C_REL), lambda i: (i, 0)),
            pl.BlockSpec((_REL_BLK, 2), lambda i: (i, 0)),
            pl.BlockSpec((_REL_BLK, 1), lambda i: (i, 0)),
        ],
        out_specs=(
            pl.BlockSpec((_REL_BLK, W), lambda i: (i, 0)),
            pl.BlockSpec((_REL_BLK, 1), lambda i: (i, 0)),
        ),
        out_shape=(
            jax.ShapeDtypeStruct((N_PAD, W), jnp.float32),
            jax.ShapeDtypeStruct((N_PAD, 1), jnp.float32),
        ),
    )(rel_pad, pair_pad, drel)


_NROW = N_PAD // 128


def _rank_body(k2d_ref, kT_ref, out_ref):
    i = pl.program_id(0)
    ki = jnp.broadcast_to(kT_ref[0], (128, 128))

    def body_ge(j, acc):
        kj = k2d_ref[pl.ds(j, 1), :]
        return acc + jnp.where(kj >= ki, 1, 0)

    def body_gt(j, acc):
        kj = k2d_ref[pl.ds(j, 1), :]
        return acc + jnp.where(kj > ki, 1, 0)

    acc = jnp.zeros((128, 128), jnp.int32)
    acc = lax.fori_loop(0, i, body_ge, acc)
    acc = lax.fori_loop(i + 1, _NROW, body_gt, acc)
    kd = k2d_ref[pl.ds(i, 1), :]
    a_ix = lax.broadcasted_iota(jnp.int32, (128, 128), 0)
    b_ix = lax.broadcasted_iota(jnp.int32, (128, 128), 1)
    acc = acc + jnp.where(kd > ki, 1, 0)
    acc = acc + jnp.where((kd == ki) & (b_ix < a_ix), 1, 0)
    out_ref[...] = jnp.sum(acc, axis=1, keepdims=True)[None]


def _tc_rank(keys2d, keys_col):
    return pl.pallas_call(
        _rank_body,
        grid=(_NROW,),
        in_specs=[
            pl.BlockSpec((_NROW, 128), lambda i: (0, 0)),
            pl.BlockSpec((1, 128, 1), lambda i: (i, 0, 0)),
        ],
        out_specs=pl.BlockSpec((1, 128, 1), lambda i: (i, 0, 0)),
        out_shape=jax.ShapeDtypeStruct((_NROW, 128, 1), jnp.int32),
    )(keys2d, keys_col)


_NW = 32
_CHUNK = N_PAD // _NW


def _sc_keys_body(scores_hbm, pairs_hbm, rels_hbm, keys_hbm,
                  scores_v, pairs_v, rels_v, keys_v):
    wid = lax.axis_index("s") * 2 + lax.axis_index("c")
    base = wid * _CHUNK
    pltpu.sync_copy(scores_hbm, scores_v)
    pltpu.sync_copy(pairs_hbm.at[pl.ds(base, _CHUNK)], pairs_v)
    pltpu.sync_copy(rels_hbm.at[pl.ds(base, _CHUNK)], rels_v)
    lanes = lax.iota(jnp.int32, 16)
    zeros = lanes * 0
    ones = zeros + 1

    def body(c, _):
        r0 = c * 16
        rows = r0 + lanes
        subj = plsc.load_gather(pairs_v, [rows, zeros])
        obj = plsc.load_gather(pairs_v, [rows, ones])
        ss = plsc.load_gather(scores_v, [subj])
        os_ = plsc.load_gather(scores_v, [obj])
        rs = rels_v[pl.ds(r0, 16)]
        key = rs * ss * os_
        valid = (base + rows) < N_REL
        keys_v[pl.ds(r0, 16)] = jnp.where(valid, key, -1.0)
        return 0

    lax.fori_loop(0, _CHUNK // 16, body, 0)
    pltpu.sync_copy(keys_v, keys_hbm.at[pl.ds(base, _CHUNK)])


def _sc_keys(pred_scores, pairs_pad, rel_scores):
    mesh = plsc.VectorSubcoreMesh(core_axis_name="c", subcore_axis_name="s")
    f = functools.partial(
        pl.kernel,
        mesh=mesh,
        compiler_params=pltpu.CompilerParams(needs_layout_passes=False),
        out_type=jax.ShapeDtypeStruct((N_PAD,), jnp.float32),
        scratch_types=[
            pltpu.VMEM((N_OBJ,), jnp.float32),
            pltpu.VMEM((_CHUNK, 2), jnp.int32),
            pltpu.VMEM((_CHUNK,), jnp.float32),
            pltpu.VMEM((_CHUNK,), jnp.float32),
        ],
    )(_sc_keys_body)
    return f(pred_scores, pairs_pad, rel_scores)


def _sc_scatter_body(comb_hbm, rank_hbm, out_hbm, rank_v, rows_v, sem):
    wid = lax.axis_index("s") * 2 + lax.axis_index("c")
    base = wid * _CHUNK
    pltpu.sync_copy(rank_hbm.at[pl.ds(base, _CHUNK)], rank_v)
    pltpu.sync_copy(comb_hbm.at[pl.ds(base, _CHUNK)], rows_v)
    pltpu.async_copy(rows_v, out_hbm.at[rank_v], sem).wait()


def _sc_scatter(comb, rank):
    mesh = plsc.VectorSubcoreMesh(core_axis_name="c", subcore_axis_name="s")
    f = functools.partial(
        pl.kernel,
        mesh=mesh,
        compiler_params=pltpu.CompilerParams(needs_layout_passes=False,
                                             use_tc_tiling_on_sc=False),
        out_type=jax.ShapeDtypeStruct((N_PAD, W), jnp.float32),
        scratch_types=[
            pltpu.VMEM((_CHUNK,), jnp.int32),
            pltpu.VMEM((_CHUNK, W), jnp.float32),
            pltpu.SemaphoreType.DMA,
        ],
    )(_sc_scatter_body)
    return f(comb, rank)


def kernel(rel_logit, obj_logit, rel_pair_idx):
    dobj = jnp.sum(jnp.exp(obj_logit - jnp.max(obj_logit, axis=1, keepdims=True)),
                   axis=1, keepdims=True)
    drel = jnp.sum(jnp.exp(rel_logit - jnp.max(rel_logit, axis=1, keepdims=True)),
                   axis=1, keepdims=True)

    score2d, label2d = _tc_obj(obj_logit, dobj)
    pred_scores = score2d[:, 0]
    pred_labels = label2d[:, 0]

    rel_pad = jnp.pad(rel_logit, ((0, N_PAD - N_REL), (0, 0)))
    pair_pad = jnp.pad(rel_pair_idx, ((0, N_PAD - N_REL), (0, 0)))
    drel_pad = jnp.pad(drel, ((0, N_PAD - N_REL), (0, 0)), constant_values=1.0)
    comb, rs2d = _tc_rel(rel_pad, pair_pad, drel_pad)
    rel_scores = rs2d[:, 0]

    keys = _sc_keys(pred_scores, pair_pad, rel_scores)

    ki = lax.bitcast_convert_type(keys, jnp.int32)
    keys2d = ki.reshape(_NROW, 128)
    rank = _tc_rank(keys2d, ki.reshape(_NROW, 128, 1)).reshape(-1)

    out = _sc_scatter(comb, rank)

    s = out[:N_REL]
    pred_rel_cls_scores = s[:, :C_REL]
    pred_rel_labels = s[:, C_REL].astype(jnp.int32)
    rel_pair_sorted = s[:, C_REL + 1:C_REL + 3].astype(jnp.int32)
    return (pred_labels, pred_scores, rel_pair_sorted,
            pred_rel_cls_scores, pred_rel_labels)


# speedup vs baseline: 1.1170x; 1.1170x over previous
import functools

import jax
import jax.numpy as jnp
from jax import lax
from jax.experimental import pallas as pl
from jax.experimental.pallas import tpu as pltpu
from jax.experimental.pallas import tpu_sc as plsc

N_REL = 20000
N_PAD = 20480
N_OBJ = 5000
C_REL = 51
C_OBJ = 151
W = 64
BIG = 10**9


def _obj_body(obj_ref, d_ref, score_ref, label_ref):
    x = obj_ref[...]
    m = jnp.max(x, axis=1, keepdims=True)
    x1 = x[:, 1:]
    m1 = jnp.max(x1, axis=1, keepdims=True)
    score_ref[...] = jnp.exp(m1 - m) / d_ref[...]
    iota = lax.broadcasted_iota(jnp.int32, x1.shape, 1)
    cand = jnp.where(x1 == m1, iota, BIG)
    label_ref[...] = jnp.min(cand, axis=1, keepdims=True) + 1


def _tc_obj(obj_logit, dobj):
    return pl.pallas_call(
        _obj_body,
        out_shape=(
            jax.ShapeDtypeStruct((N_OBJ, 1), jnp.float32),
            jax.ShapeDtypeStruct((N_OBJ, 1), jnp.int32),
        ),
    )(obj_logit, dobj)


_REL_BLK = 2048


def _rel_body(rel_ref, pair_ref, d_ref, comb_ref, rs_ref):
    x = rel_ref[...]
    m = jnp.max(x, axis=1, keepdims=True)
    e = jnp.exp(x - m)
    p = e / d_ref[...]
    rs_ref[...] = jnp.max(p[:, 1:], axis=1, keepdims=True)
    pm = jnp.max(p, axis=1, keepdims=True)
    iota = lax.broadcasted_iota(jnp.int32, p.shape, 1)
    cls = jnp.min(jnp.where(p == pm, iota, BIG), axis=1, keepdims=True)
    pairf = pair_ref[...].astype(jnp.float32)
    comb_ref[...] = jnp.concatenate(
        [p, cls.astype(jnp.float32), pairf,
         jnp.zeros((x.shape[0], W - C_REL - 3), jnp.float32)], axis=1)


def _tc_rel(rel_pad, pair_pad, drel):
    grid = N_PAD // _REL_BLK
    return pl.pallas_call(
        _rel_body,
        grid=(grid,),
        in_specs=[
            pl.BlockSpec((_REL_BLK, C_REL), lambda i: (i, 0)),
            pl.BlockSpec((_REL_BLK, 2), lambda i: (i, 0)),
            pl.BlockSpec((_REL_BLK, 1), lambda i: (i, 0)),
        ],
        out_specs=(
            pl.BlockSpec((_REL_BLK, W), lambda i: (i, 0)),
            pl.BlockSpec((_REL_BLK, 1), lambda i: (i, 0)),
        ),
        out_shape=(
            jax.ShapeDtypeStruct((N_PAD, W), jnp.float32),
            jax.ShapeDtypeStruct((N_PAD, 1), jnp.float32),
        ),
    )(rel_pad, pair_pad, drel)


_NROW = N_PAD // 128


def _rank_body(k2d_ref, kT_ref, out_ref):
    i = pl.program_id(0)
    ki = jnp.broadcast_to(kT_ref[0], (128, 128))

    def body_ge(j8, acc):
        kj8 = k2d_ref[pl.ds(j8 * 8, 8), :]
        for r in range(8):
            acc = acc + jnp.where(kj8[r:r + 1] >= ki, 1, 0)
        return acc

    def body_gt(j8, acc):
        kj8 = k2d_ref[pl.ds(j8 * 8, 8), :]
        for r in range(8):
            acc = acc + jnp.where(kj8[r:r + 1] > ki, 1, 0)
        return acc

    g = i // 8
    acc = jnp.zeros((128, 128), jnp.int32)
    acc = lax.fori_loop(0, g, body_ge, acc)
    acc = lax.fori_loop(g + 1, _NROW // 8, body_gt, acc)
    for r in range(8):
        j = g * 8 + r
        kj = k2d_ref[pl.ds(j, 1), :]
        ge = jnp.where(j < i, jnp.where(kj >= ki, 1, 0), 0)
        gt = jnp.where(j > i, jnp.where(kj > ki, 1, 0), 0)
        acc = acc + ge + gt
    kd = k2d_ref[pl.ds(i, 1), :]
    a_ix = lax.broadcasted_iota(jnp.int32, (128, 128), 0)
    b_ix = lax.broadcasted_iota(jnp.int32, (128, 128), 1)
    acc = acc + jnp.where(kd > ki, 1, 0)
    acc = acc + jnp.where((kd == ki) & (b_ix < a_ix), 1, 0)
    out_ref[...] = jnp.sum(acc, axis=1, keepdims=True)[None]


def _tc_rank(keys2d, keys_col):
    return pl.pallas_call(
        _rank_body,
        grid=(_NROW,),
        in_specs=[
            pl.BlockSpec((_NROW, 128), lambda i: (0, 0)),
            pl.BlockSpec((1, 128, 1), lambda i: (i, 0, 0)),
        ],
        out_specs=pl.BlockSpec((1, 128, 1), lambda i: (i, 0, 0)),
        out_shape=jax.ShapeDtypeStruct((_NROW, 128, 1), jnp.int32),
    )(keys2d, keys_col)


_NW = 32
_CHUNK = N_PAD // _NW


def _sc_keys_body(scores_hbm, pairs_hbm, rels_hbm, keys_hbm,
                  scores_v, pairs_v, rels_v, keys_v):
    wid = lax.axis_index("s") * 2 + lax.axis_index("c")
    base = wid * _CHUNK
    pltpu.sync_copy(scores_hbm, scores_v)
    pltpu.sync_copy(pairs_hbm.at[pl.ds(base, _CHUNK)], pairs_v)
    pltpu.sync_copy(rels_hbm.at[pl.ds(base, _CHUNK)], rels_v)
    lanes = lax.iota(jnp.int32, 16)
    zeros = lanes * 0
    ones = zeros + 1

    def body(c, _):
        r0 = c * 16
        rows = r0 + lanes
        subj = plsc.load_gather(pairs_v, [rows, zeros])
        obj = plsc.load_gather(pairs_v, [rows, ones])
        ss = plsc.load_gather(scores_v, [subj])
        os_ = plsc.load_gather(scores_v, [obj])
        rs = rels_v[pl.ds(r0, 16)]
        key = rs * ss * os_
        valid = (base + rows) < N_REL
        keys_v[pl.ds(r0, 16)] = jnp.where(valid, key, -1.0)
        return 0

    lax.fori_loop(0, _CHUNK // 16, body, 0)
    pltpu.sync_copy(keys_v, keys_hbm.at[pl.ds(base, _CHUNK)])


def _sc_keys(pred_scores, pairs_pad, rel_scores):
    mesh = plsc.VectorSubcoreMesh(core_axis_name="c", subcore_axis_name="s")
    f = functools.partial(
        pl.kernel,
        mesh=mesh,
        compiler_params=pltpu.CompilerParams(needs_layout_passes=False),
        out_type=jax.ShapeDtypeStruct((N_PAD,), jnp.float32),
        scratch_types=[
            pltpu.VMEM((N_OBJ,), jnp.float32),
            pltpu.VMEM((_CHUNK, 2), jnp.int32),
            pltpu.VMEM((_CHUNK,), jnp.float32),
            pltpu.VMEM((_CHUNK,), jnp.float32),
        ],
    )(_sc_keys_body)
    return f(pred_scores, pairs_pad, rel_scores)


def _sc_scatter_body(comb_hbm, rank_hbm, out_hbm, rank_v, rows_v, sem):
    wid = lax.axis_index("s") * 2 + lax.axis_index("c")
    base = wid * _CHUNK
    pltpu.sync_copy(rank_hbm.at[pl.ds(base, _CHUNK)], rank_v)
    pltpu.sync_copy(comb_hbm.at[pl.ds(base, _CHUNK)], rows_v)
    pltpu.async_copy(rows_v, out_hbm.at[rank_v], sem).wait()


def _sc_scatter(comb, rank):
    mesh = plsc.VectorSubcoreMesh(core_axis_name="c", subcore_axis_name="s")
    f = functools.partial(
        pl.kernel,
        mesh=mesh,
        compiler_params=pltpu.CompilerParams(needs_layout_passes=False,
                                             use_tc_tiling_on_sc=False),
        out_type=jax.ShapeDtypeStruct((N_PAD, W), jnp.float32),
        scratch_types=[
            pltpu.VMEM((_CHUNK,), jnp.int32),
            pltpu.VMEM((_CHUNK, W), jnp.float32),
            pltpu.SemaphoreType.DMA,
        ],
    )(_sc_scatter_body)
    return f(comb, rank)


def kernel(rel_logit, obj_logit, rel_pair_idx):
    dobj = jnp.sum(jnp.exp(obj_logit - jnp.max(obj_logit, axis=1, keepdims=True)),
                   axis=1, keepdims=True)
    drel = jnp.sum(jnp.exp(rel_logit - jnp.max(rel_logit, axis=1, keepdims=True)),
                   axis=1, keepdims=True)

    score2d, label2d = _tc_obj(obj_logit, dobj)
    pred_scores = score2d[:, 0]
    pred_labels = label2d[:, 0]

    rel_pad = jnp.pad(rel_logit, ((0, N_PAD - N_REL), (0, 0)))
    pair_pad = jnp.pad(rel_pair_idx, ((0, N_PAD - N_REL), (0, 0)))
    drel_pad = jnp.pad(drel, ((0, N_PAD - N_REL), (0, 0)), constant_values=1.0)
    comb, rs2d = _tc_rel(rel_pad, pair_pad, drel_pad)
    rel_scores = rs2d[:, 0]

    keys = _sc_keys(pred_scores, pair_pad, rel_scores)

    ki = lax.bitcast_convert_type(keys, jnp.int32)
    keys2d = ki.reshape(_NROW, 128)
    rank = _tc_rank(keys2d, ki.reshape(_NROW, 128, 1)).reshape(-1)

    out = _sc_scatter(comb, rank)

    s = out[:N_REL]
    pred_rel_cls_scores = s[:, :C_REL]
    pred_rel_labels = s[:, C_REL].astype(jnp.int32)
    rel_pair_sorted = s[:, C_REL + 1:C_REL + 3].astype(jnp.int32)
    return (pred_labels, pred_scores, rel_pair_sorted,
            pred_rel_cls_scores, pred_rel_labels)


# speedup vs baseline: 2.2307x; 1.9971x over previous
import functools

import jax
import jax.numpy as jnp
from jax import lax
from jax.experimental import pallas as pl
from jax.experimental.pallas import tpu as pltpu
from jax.experimental.pallas import tpu_sc as plsc

N_REL = 20000
N_PAD = 20480
N_OBJ = 5000
C_REL = 51
C_OBJ = 151
W = 64
BIG = 10**9


def _obj_body(obj_ref, d_ref, score_ref, label_ref):
    x = obj_ref[...]
    m = jnp.max(x, axis=1, keepdims=True)
    x1 = x[:, 1:]
    m1 = jnp.max(x1, axis=1, keepdims=True)
    score_ref[...] = jnp.exp(m1 - m) / d_ref[...]
    iota = lax.broadcasted_iota(jnp.int32, x1.shape, 1)
    cand = jnp.where(x1 == m1, iota, BIG)
    label_ref[...] = jnp.min(cand, axis=1, keepdims=True) + 1


def _tc_obj(obj_logit, dobj):
    return pl.pallas_call(
        _obj_body,
        out_shape=(
            jax.ShapeDtypeStruct((N_OBJ, 1), jnp.float32),
            jax.ShapeDtypeStruct((N_OBJ, 1), jnp.int32),
        ),
    )(obj_logit, dobj)


_REL_BLK = 2048


def _rel_body(rel_ref, pair_ref, d_ref, comb_ref, rs_ref):
    x = rel_ref[...]
    m = jnp.max(x, axis=1, keepdims=True)
    e = jnp.exp(x - m)
    p = e / d_ref[...]
    rs_ref[...] = jnp.max(p[:, 1:], axis=1, keepdims=True)
    pm = jnp.max(p, axis=1, keepdims=True)
    iota = lax.broadcasted_iota(jnp.int32, p.shape, 1)
    cls = jnp.min(jnp.where(p == pm, iota, BIG), axis=1, keepdims=True)
    pairf = pair_ref[...].astype(jnp.float32)
    comb_ref[...] = jnp.concatenate(
        [p, cls.astype(jnp.float32), pairf,
         jnp.zeros((x.shape[0], W - C_REL - 3), jnp.float32)], axis=1)


def _tc_rel(rel_pad, pair_pad, drel):
    grid = N_PAD // _REL_BLK
    return pl.pallas_call(
        _rel_body,
        grid=(grid,),
        in_specs=[
            pl.BlockSpec((_REL_BLK, C_REL), lambda i: (i, 0)),
            pl.BlockSpec((_REL_BLK, 2), lambda i: (i, 0)),
            pl.BlockSpec((_REL_BLK, 1), lambda i: (i, 0)),
        ],
        out_specs=(
            pl.BlockSpec((_REL_BLK, W), lambda i: (i, 0)),
            pl.BlockSpec((_REL_BLK, 1), lambda i: (i, 0)),
        ),
        out_shape=(
            jax.ShapeDtypeStruct((N_PAD, W), jnp.float32),
            jax.ShapeDtypeStruct((N_PAD, 1), jnp.float32),
        ),
    )(rel_pad, pair_pad, drel)


_NROW = N_PAD // 128


def _rank_body(k2d_ref, kT_ref, out_ref):
    i = pl.program_id(0)
    ki = jnp.broadcast_to(kT_ref[0], (128, 128))

    def body_ge(j8, acc):
        kj8 = k2d_ref[pl.ds(j8 * 8, 8), :]
        for r in range(8):
            acc = acc + jnp.where(kj8[r:r + 1] >= ki, 1, 0)
        return acc

    def body_gt(j8, acc):
        kj8 = k2d_ref[pl.ds(j8 * 8, 8), :]
        for r in range(8):
            acc = acc + jnp.where(kj8[r:r + 1] > ki, 1, 0)
        return acc

    g = i // 8
    acc = jnp.zeros((128, 128), jnp.int32)
    acc = lax.fori_loop(0, g, body_ge, acc)
    acc = lax.fori_loop(g + 1, _NROW // 8, body_gt, acc)
    for r in range(8):
        j = g * 8 + r
        kj = k2d_ref[pl.ds(j, 1), :]
        ge = jnp.where(j < i, jnp.where(kj >= ki, 1, 0), 0)
        gt = jnp.where(j > i, jnp.where(kj > ki, 1, 0), 0)
        acc = acc + ge + gt
    kd = k2d_ref[pl.ds(i, 1), :]
    a_ix = lax.broadcasted_iota(jnp.int32, (128, 128), 0)
    b_ix = lax.broadcasted_iota(jnp.int32, (128, 128), 1)
    acc = acc + jnp.where(kd > ki, 1, 0)
    acc = acc + jnp.where((kd == ki) & (b_ix < a_ix), 1, 0)
    out_ref[...] = jnp.sum(acc, axis=1, keepdims=True)[None]


def _tc_rank(keys2d, keys_col):
    return pl.pallas_call(
        _rank_body,
        grid=(_NROW,),
        in_specs=[
            pl.BlockSpec((_NROW, 128), lambda i: (0, 0)),
            pl.BlockSpec((1, 128, 1), lambda i: (i, 0, 0)),
        ],
        out_specs=pl.BlockSpec((1, 128, 1), lambda i: (i, 0, 0)),
        out_shape=jax.ShapeDtypeStruct((_NROW, 128, 1), jnp.int32),
    )(keys2d, keys_col)


_NW = 32
_CHUNK = N_PAD // _NW


def _sc_keys_body(scores_hbm, pairs_hbm, rels_hbm, keys_hbm,
                  scores_v, pairs_v, rels_v, keys_v):
    wid = lax.axis_index("s") * 2 + lax.axis_index("c")
    base = wid * _CHUNK
    pltpu.sync_copy(scores_hbm, scores_v)
    pltpu.sync_copy(pairs_hbm.at[pl.ds(base, _CHUNK)], pairs_v)
    pltpu.sync_copy(rels_hbm.at[pl.ds(base, _CHUNK)], rels_v)
    lanes = lax.iota(jnp.int32, 16)
    zeros = lanes * 0
    ones = zeros + 1

    def body(c, _):
        r0 = c * 16
        rows = r0 + lanes
        subj = plsc.load_gather(pairs_v, [rows, zeros])
        obj = plsc.load_gather(pairs_v, [rows, ones])
        ss = plsc.load_gather(scores_v, [subj])
        os_ = plsc.load_gather(scores_v, [obj])
        rs = rels_v[pl.ds(r0, 16)]
        key = rs * ss * os_
        valid = (base + rows) < N_REL
        keys_v[pl.ds(r0, 16)] = jnp.where(valid, key, -1.0)
        return 0

    lax.fori_loop(0, _CHUNK // 16, body, 0)
    pltpu.sync_copy(keys_v, keys_hbm.at[pl.ds(base, _CHUNK)])


def _sc_keys(pred_scores, pairs_pad, rel_scores):
    mesh = plsc.VectorSubcoreMesh(core_axis_name="c", subcore_axis_name="s")
    f = functools.partial(
        pl.kernel,
        mesh=mesh,
        compiler_params=pltpu.CompilerParams(needs_layout_passes=False),
        out_type=jax.ShapeDtypeStruct((N_PAD,), jnp.float32),
        scratch_types=[
            pltpu.VMEM((N_OBJ,), jnp.float32),
            pltpu.VMEM((_CHUNK, 2), jnp.int32),
            pltpu.VMEM((_CHUNK,), jnp.float32),
            pltpu.VMEM((_CHUNK,), jnp.float32),
        ],
    )(_sc_keys_body)
    return f(pred_scores, pairs_pad, rel_scores)


def _sc_scatter_body(comb_hbm, rank_hbm, out_hbm, rank_v, rows_v, sem):
    wid = lax.axis_index("s") * 2 + lax.axis_index("c")
    base = wid * _CHUNK
    pltpu.sync_copy(rank_hbm.at[pl.ds(base, _CHUNK)], rank_v)
    pltpu.sync_copy(comb_hbm.at[pl.ds(base, _CHUNK)], rows_v)
    pltpu.async_copy(rows_v, out_hbm.at[rank_v], sem).wait()


def _sc_scatter(comb, rank):
    mesh = plsc.VectorSubcoreMesh(core_axis_name="c", subcore_axis_name="s")
    f = functools.partial(
        pl.kernel,
        mesh=mesh,
        compiler_params=pltpu.CompilerParams(needs_layout_passes=False,
                                             use_tc_tiling_on_sc=False),
        out_type=jax.ShapeDtypeStruct((N_PAD, W), jnp.float32),
        scratch_types=[
            pltpu.VMEM((_CHUNK,), jnp.int32),
            pltpu.VMEM((_CHUNK, W), jnp.float32),
            pltpu.SemaphoreType.DMA,
        ],
    )(_sc_scatter_body)
    return f(comb, rank)


_NWS = 16
_CH = N_PAD // _NWS
_NCK = _CH // 16


def _sc_sort_body(scores_hbm, pairs_hbm, rels_hbm, comb_hbm, out_hbm,
                  scores_v, pairs_v, rels_v, kv, iv, dest_v, hist_v, offs_v,
                  allh_v, rows_v, sem,
                  hists_s, bufa_k, bufa_i, bufb_k, bufb_i):
    wid = lax.axis_index("s")
    base = wid * _CH
    lanes = lax.iota(jnp.int32, 16)
    zeros = lanes * 0
    ones = zeros + 1

    pltpu.sync_copy(scores_hbm, scores_v)
    pltpu.sync_copy(pairs_hbm.at[pl.ds(base, _CH)], pairs_v)
    pltpu.sync_copy(rels_hbm.at[pl.ds(base, _CH)], rels_v)

    def kbody(c, _):
        r0 = c * 16
        rows = r0 + lanes
        subj = plsc.load_gather(pairs_v, [rows, zeros])
        obj = plsc.load_gather(pairs_v, [rows, ones])
        ss = plsc.load_gather(scores_v, [subj])
        os_ = plsc.load_gather(scores_v, [obj])
        key = rels_v[pl.ds(r0, 16)] * ss * os_
        valid = (base + rows) < N_REL
        keyf = jnp.where(valid, key, -1.0)
        ku = plsc.bitcast(keyf, jnp.uint32)
        kv[pl.ds(r0, 16)] = jnp.uint32(0x7FFFFFFF) - ku
        iv[pl.ds(r0, 16)] = base + rows
        return 0

    lax.fori_loop(0, _NCK, kbody, 0)
    pltpu.sync_copy(kv, bufa_k.at[pl.ds(base, _CH)])
    pltpu.sync_copy(iv, bufa_i.at[pl.ds(base, _CH)])
    plsc.subcore_barrier()

    for p in range(4):
        src_k, src_i = (bufa_k, bufa_i) if p % 2 == 0 else (bufb_k, bufb_i)
        dst_k, dst_i = (bufb_k, bufb_i) if p % 2 == 0 else (bufa_k, bufa_i)
        shift = jnp.uint32(8 * p)
        pltpu.sync_copy(src_k.at[pl.ds(base, _CH)], kv)
        pltpu.sync_copy(src_i.at[pl.ds(base, _CH)], iv)
        for c in range(16):
            hist_v[pl.ds(c * 16, 16)] = zeros

        def hbody(c, _):
            d = ((kv[pl.ds(c * 16, 16)] >> shift) & jnp.uint32(255)
                 ).astype(jnp.int32)
            cnt, last = plsc.scan_count(d)
            plsc.addupdate_scatter(hist_v, [d], cnt, mask=last)
            return 0

        lax.fori_loop(0, _NCK, hbody, 0)
        pltpu.sync_copy(hist_v, hists_s.at[wid])
        plsc.subcore_barrier()
        pltpu.sync_copy(hists_s, allh_v)

        carry = jnp.int32(0)
        for c in range(16):
            def wbody(w, tb):
                tot, before = tb
                h = allh_v[w, pl.ds(c * 16, 16)]
                return (tot + h, before + jnp.where(w < wid, h, 0))

            tot, before = lax.fori_loop(
                0, _NWS, wbody,
                (jnp.zeros((16,), jnp.int32), jnp.zeros((16,), jnp.int32)))
            csum = plsc.cumsum(tot)
            offs_v[pl.ds(c * 16, 16)] = carry + (csum - tot) + before
            carry = carry + jnp.sum(tot)

        def pbody(c, _):
            d = ((kv[pl.ds(c * 16, 16)] >> shift) & jnp.uint32(255)
                 ).astype(jnp.int32)
            cnt, last = plsc.scan_count(d)
            off = plsc.load_gather(offs_v, [d])
            dest = off + (cnt - 1)
            plsc.store_scatter(offs_v, [d], dest + 1, mask=last)
            dest_v[pl.ds(c * 16, 16)] = dest
            return 0

        lax.fori_loop(0, _NCK, pbody, 0)
        pltpu.sync_copy(kv, dst_k.at[dest_v])
        pltpu.sync_copy(iv, dst_i.at[dest_v])
        plsc.subcore_barrier()

    pltpu.sync_copy(bufa_i.at[pl.ds(base, _CH)], iv)
    pltpu.async_copy(comb_hbm.at[iv], rows_v, sem).wait()
    pltpu.sync_copy(rows_v, out_hbm.at[pl.ds(base, _CH)])


def _sc_sort(pred_scores, pairs_pad, rel_scores, comb):
    mesh = plsc.VectorSubcoreMesh(core_axis_name="c", subcore_axis_name="s",
                                  num_cores=1)
    f = functools.partial(
        pl.kernel,
        mesh=mesh,
        compiler_params=pltpu.CompilerParams(needs_layout_passes=False,
                                             use_tc_tiling_on_sc=False),
        out_type=jax.ShapeDtypeStruct((N_PAD, W), jnp.float32),
        scratch_types=[
            pltpu.VMEM((N_OBJ,), jnp.float32),
            pltpu.VMEM((_CH, 2), jnp.int32),
            pltpu.VMEM((_CH,), jnp.float32),
            pltpu.VMEM((_CH,), jnp.uint32),
            pltpu.VMEM((_CH,), jnp.int32),
            pltpu.VMEM((_CH,), jnp.int32),
            pltpu.VMEM((256,), jnp.int32),
            pltpu.VMEM((256,), jnp.int32),
            pltpu.VMEM((_NWS, 256), jnp.int32),
            pltpu.VMEM((_CH, W), jnp.float32),
            pltpu.SemaphoreType.DMA,
            pltpu.VMEM_SHARED((_NWS, 256), jnp.int32),
            pltpu.VMEM_SHARED((N_PAD,), jnp.uint32),
            pltpu.VMEM_SHARED((N_PAD,), jnp.int32),
            pltpu.VMEM_SHARED((N_PAD,), jnp.uint32),
            pltpu.VMEM_SHARED((N_PAD,), jnp.int32),
        ],
    )(_sc_sort_body)
    return f(pred_scores, pairs_pad, rel_scores, comb)


def kernel(rel_logit, obj_logit, rel_pair_idx):
    dobj = jnp.sum(jnp.exp(obj_logit - jnp.max(obj_logit, axis=1, keepdims=True)),
                   axis=1, keepdims=True)
    drel = jnp.sum(jnp.exp(rel_logit - jnp.max(rel_logit, axis=1, keepdims=True)),
                   axis=1, keepdims=True)

    score2d, label2d = _tc_obj(obj_logit, dobj)
    pred_scores = score2d[:, 0]
    pred_labels = label2d[:, 0]

    rel_pad = jnp.pad(rel_logit, ((0, N_PAD - N_REL), (0, 0)))
    pair_pad = jnp.pad(rel_pair_idx, ((0, N_PAD - N_REL), (0, 0)))
    drel_pad = jnp.pad(drel, ((0, N_PAD - N_REL), (0, 0)), constant_values=1.0)
    comb, rs2d = _tc_rel(rel_pad, pair_pad, drel_pad)
    rel_scores = rs2d[:, 0]

    out = _sc_sort(pred_scores, pair_pad, rel_scores, comb)

    s = out[:N_REL]
    pred_rel_cls_scores = s[:, :C_REL]
    pred_rel_labels = s[:, C_REL].astype(jnp.int32)
    rel_pair_sorted = s[:, C_REL + 1:C_REL + 3].astype(jnp.int32)
    return (pred_labels, pred_scores, rel_pair_sorted,
            pred_rel_cls_scores, pred_rel_labels)
